# baseline (speedup 1.0000x reference)
import jax
import jax.numpy as jnp
from jax import lax
from jax.experimental import pallas as pl
from jax.experimental.pallas import tpu as pltpu
from jax.experimental.pallas import tpu_sc as plsc

N = 10000
E = 320000
F = 128
H = 128
C = 40
CP = 48
NPAD = 10240
NC = 2
NS = 16
NW = NC * NS
B = 80
STEPS = 128
CH = 16
NBUF = 4
EPW = STEPS * B
ROWS_PER_TILE = NPAD // NS


def _make_agg(D, tc_tiling=True):
    mesh = plsc.VectorSubcoreMesh(core_axis_name="c", subcore_axis_name="s")
    out = jax.ShapeDtypeStruct((NC * NPAD, D), jnp.float32)
    scratch = [
        pltpu.VMEM((CH, B), jnp.int32),
        pltpu.VMEM((CH, B), jnp.int32),
        pltpu.VMEM((CH, B), jnp.int32),
        pltpu.VMEM((CH, B), jnp.int32),
        pltpu.VMEM((NBUF, B, D), jnp.float32),
        pltpu.VMEM_SHARED((NPAD, D), jnp.float32),
        pltpu.SemaphoreType.DMA,
        pltpu.SemaphoreType.DMA,
        pltpu.SemaphoreType.DMA,
        pltpu.SemaphoreType.DMA,
        pltpu.SemaphoreType.DMA,
    ]

    def body(src_hbm, dst_hbm, x_hbm, z_hbm, out_hbm,
             srcA, dstA, srcB, dstB, rows_v, acc_sh,
             gsem0, gsem1, ssem0, ssem1, isem):
        gsems = [gsem0, gsem1, ssem0, ssem1]
        ssems = gsems
        c = lax.axis_index("c")
        s = lax.axis_index("s")
        wid = s * NC + c
        row0 = s * ROWS_PER_TILE

        pltpu.sync_copy(z_hbm, rows_v.at[0])
        for i in range(ROWS_PER_TILE // B):
            pltpu.sync_copy(rows_v.at[0], acc_sh.at[pl.ds(row0 + i * B, B)])
        plsc.subcore_barrier()

        def gather(sv, j, buf):
            pltpu.async_copy(x_hbm.at[sv.at[j]], rows_v.at[buf], gsems[buf])

        def gwait(sv, j, buf):
            pltpu.make_async_copy(x_hbm.at[sv.at[j]], rows_v.at[buf],
                                  gsems[buf]).wait()

        def sfire(dv, j, buf):
            pltpu.async_copy(rows_v.at[buf], acc_sh.at[dv.at[j]],
                             ssems[buf], add=True)

        def swait(dv, j, buf):
            pltpu.make_async_copy(rows_v.at[buf], acc_sh.at[dv.at[j]],
                                  ssems[buf]).wait()

        def iwait(dst_ref):
            pltpu.make_async_copy(src_hbm.at[wid, pl.ds(0, CH)],
                                  dst_ref, isem).wait()

        NITER = STEPS // (2 * CH)

        def half(i, sv, dv, first):
            for p in range(CH):
                b = p % NBUF
                gwait(sv, p, b)
                nxt = p + NBUF
                if nxt < CH:
                    gather(sv, nxt, b)
                elif first:
                    if nxt == CH:
                        iwait(srcB)
                        iwait(dstB)
                    gather(srcB, nxt - CH, b)
                else:
                    @pl.when(i + 1 < NITER)
                    def _():
                        if nxt == CH:
                            iwait(srcA)
                            iwait(dstA)
                        gather(srcA, nxt - CH, b)

        def iteration(i, carry):
            cA = 2 * i
            cB = 2 * i + 1
            half(i, srcA, dstA, True)
            @pl.when(i + 1 < NITER)
            def _():
                pltpu.async_copy(
                    src_hbm.at[wid, pl.ds((cA + 2) * CH, CH)], srcA, isem)
                pltpu.async_copy(
                    dst_hbm.at[wid, pl.ds((cA + 2) * CH, CH)], dstA, isem)
            half(i, srcB, dstB, False)
            @pl.when(i + 1 < NITER)
            def _():
                pltpu.async_copy(
                    src_hbm.at[wid, pl.ds((cB + 2) * CH, CH)], srcB, isem)
                pltpu.async_copy(
                    dst_hbm.at[wid, pl.ds((cB + 2) * CH, CH)], dstB, isem)
            return carry

        pltpu.sync_copy(src_hbm.at[wid, pl.ds(0, CH)], srcA)
        pltpu.sync_copy(dst_hbm.at[wid, pl.ds(0, CH)], dstA)
        pltpu.async_copy(src_hbm.at[wid, pl.ds(CH, CH)], srcB, isem)
        pltpu.async_copy(dst_hbm.at[wid, pl.ds(CH, CH)], dstB, isem)
        for b in range(NBUF):
            gather(srcA, b, b)
        lax.fori_loop(0, NITER, iteration, 0)

        plsc.subcore_barrier()
        out_base = c * NPAD + row0
        for i in range(ROWS_PER_TILE // B):
            pltpu.sync_copy(acc_sh.at[pl.ds(row0 + i * B, B)], rows_v.at[0])
            pltpu.sync_copy(rows_v.at[0],
                            out_hbm.at[pl.ds(out_base + i * B, B)])

    return pl.kernel(
        body, out_type=out, mesh=mesh, scratch_types=scratch,
        compiler_params=pltpu.CompilerParams(use_tc_tiling_on_sc=tc_tiling))


def _make_deg():
    mesh = plsc.VectorSubcoreMesh(core_axis_name="c", subcore_axis_name="s")
    out = jax.ShapeDtypeStruct((NC * NPAD, 16), jnp.float32)
    scratch = [
        pltpu.VMEM((CH, B), jnp.int32),
        pltpu.VMEM((B, 16), jnp.float32),
        pltpu.VMEM_SHARED((NPAD, 16), jnp.float32),
    ]

    def body(dst_hbm, z16_hbm, ones_hbm, deg_hbm, dst_v, uno_v, deg_sh):
        c = lax.axis_index("c")
        s = lax.axis_index("s")
        wid = s * NC + c
        row0 = s * ROWS_PER_TILE

        pltpu.sync_copy(z16_hbm, uno_v)
        for i in range(ROWS_PER_TILE // B):
            pltpu.sync_copy(uno_v, deg_sh.at[pl.ds(row0 + i * B, B)])
        pltpu.sync_copy(ones_hbm, uno_v)
        plsc.subcore_barrier()

        def chunk_step(k, carry):
            pltpu.sync_copy(dst_hbm.at[wid, pl.ds(k * CH, CH)], dst_v)
            for jj in range(CH):
                pltpu.sync_copy(uno_v, deg_sh.at[dst_v.at[jj]], add=True)
            return carry
        lax.fori_loop(0, STEPS // CH, chunk_step, 0)

        plsc.subcore_barrier()
        out_base = c * NPAD + row0
        for i in range(ROWS_PER_TILE // B):
            pltpu.sync_copy(deg_sh.at[pl.ds(row0 + i * B, B)], uno_v)
            pltpu.sync_copy(uno_v, deg_hbm.at[pl.ds(out_base + i * B, B)])

    return pl.kernel(
        body, out_type=out, mesh=mesh, scratch_types=scratch,
        compiler_params=pltpu.CompilerParams(use_tc_tiling_on_sc=False))


_agg_x = _make_agg(F)
_agg_g = _make_agg(CP, tc_tiling=False)
_deg = _make_deg()


BN = 1024


def _tc1_body(p_ref, d_ref, w_ref, b_ref, o_ref):
    inv = 1.0 / jnp.maximum(d_ref[0, :, 0] + d_ref[1, :, 0], 1.0)
    h = (p_ref[0] + p_ref[1]) * inv[:, None]
    y = jnp.dot(h, w_ref[...], preferred_element_type=jnp.float32,
                precision=lax.Precision.HIGHEST)
    o_ref[...] = jnp.maximum(y + b_ref[...], 0.0)


def _tc2_body(p_ref, d_ref, w1_ref, b1_ref, w2_ref, o_ref):
    inv = 1.0 / jnp.maximum(d_ref[0, :, 0] + d_ref[1, :, 0], 1.0)
    h = (p_ref[0] + p_ref[1]) * inv[:, None]
    y = jnp.dot(h, w1_ref[...], preferred_element_type=jnp.float32,
                precision=lax.Precision.HIGHEST)
    y = jnp.maximum(y + b1_ref[...], 0.0)
    o_ref[...] = jnp.dot(y, w2_ref[...], preferred_element_type=jnp.float32,
                         precision=lax.Precision.HIGHEST)


def _tc3_body(p_ref, d_ref, b_ref, o_ref):
    inv = 1.0 / jnp.maximum(d_ref[0, :, 0] + d_ref[1, :, 0], 1.0)
    z = (p_ref[0] + p_ref[1]) * inv[:, None] + b_ref[...]
    mask = lax.broadcasted_iota(jnp.int32, (BN, CP), 1) < C
    zm = jnp.where(mask, z, -jnp.inf)
    m = jnp.max(zm, axis=1, keepdims=True)
    e = jnp.where(mask, jnp.exp(z - m), 0.0)
    lse = jnp.log(jnp.sum(e, axis=1, keepdims=True))
    o_ref[...] = z - m - lse


def _pspec(d):
    return pl.BlockSpec((2, BN, d), lambda i: (0, i, 0))


def _dspec():
    return pl.BlockSpec((2, BN, 16), lambda i: (0, i, 0))


def _full(shape):
    nd = len(shape)
    return pl.BlockSpec(shape, lambda i: (0,) * nd)


def _tc1(p, d, w, b):
    return pl.pallas_call(
        _tc1_body,
        grid=(NPAD // BN,),
        in_specs=[_pspec(H), _dspec(), _full((F, H)), _full((1, H))],
        out_specs=pl.BlockSpec((BN, H), lambda i: (i, 0)),
        out_shape=jax.ShapeDtypeStruct((NPAD, H), jnp.float32),
    )(p, d, w, b)


def _tc2(p, d, w1, b1, w2):
    return pl.pallas_call(
        _tc2_body,
        grid=(NPAD // BN,),
        in_specs=[_pspec(H), _dspec(), _full((H, H)), _full((1, H)),
                  _full((H, CP))],
        out_specs=pl.BlockSpec((BN, CP), lambda i: (i, 0)),
        out_shape=jax.ShapeDtypeStruct((NPAD, CP), jnp.float32),
    )(p, d, w1, b1, w2)


def _tc3(p, d, b):
    return pl.pallas_call(
        _tc3_body,
        grid=(NPAD // BN,),
        in_specs=[_pspec(CP), _dspec(), _full((1, CP))],
        out_specs=pl.BlockSpec((BN, CP), lambda i: (i, 0)),
        out_shape=jax.ShapeDtypeStruct((NPAD, CP), jnp.float32),
    )(p, d, b)


def kernel(features, edge_index, W0, b0, W1, b1, W2, b2):
    ei = edge_index.astype(jnp.int32)
    pad = NW * EPW - E
    src = jnp.concatenate([ei[0], jnp.full((pad,), N, jnp.int32)])
    dst = jnp.concatenate([ei[1], jnp.full((pad,), N, jnp.int32)])
    src = src.reshape(NW, STEPS, B)
    dst = dst.reshape(NW, STEPS, B)

    x = jnp.zeros((NPAD, F), jnp.float32).at[:N].set(features)
    zF = jnp.zeros((B, F), jnp.float32)
    zCP = jnp.zeros((B, CP), jnp.float32)
    z16 = jnp.zeros((B, 16), jnp.float32)
    ones16 = jnp.ones((B, 16), jnp.float32)

    w2p = jnp.zeros((H, CP), jnp.float32).at[:, :C].set(W2)
    b2p = jnp.zeros((1, CP), jnp.float32).at[0, :C].set(b2)

    deg = _deg(dst, z16, ones16).reshape(NC, NPAD, 16)
    p0 = _agg_x(src, dst, x, zF).reshape(NC, NPAD, F)

    h1 = _tc1(p0, deg, W0, b0.reshape(1, H))
    q = _agg_x(src, dst, h1, zF).reshape(NC, NPAD, H)
    g = _tc2(q, deg, W1, b1.reshape(1, H), w2p)
    r = _agg_g(src, dst, g, zCP).reshape(NC, NPAD, CP)
    z = _tc3(r, deg, b2p)
    return z[:N, :C]


# speedup vs baseline: 1.2731x; 1.2731x over previous
import jax
import jax.numpy as jnp
from jax import lax
from jax.experimental import pallas as pl
from jax.experimental.pallas import tpu as pltpu
from jax.experimental.pallas import tpu_sc as plsc

N = 10000
E = 320000
F = 128
H = 128
C = 40
CP = 48
NPAD = 10240
NC = 2
NS = 16
NW = NC * NS
B = 128
STEPS = 80
CH = 8
NBUF = 2
EPW = STEPS * B
ROWS_PER_TILE = NPAD // NS


def _make_agg(D, tc_tiling=True):
    mesh = plsc.VectorSubcoreMesh(core_axis_name="c", subcore_axis_name="s")
    out = jax.ShapeDtypeStruct((NC * NPAD, D), jnp.float32)
    scratch = [
        pltpu.VMEM((CH, B), jnp.int32),
        pltpu.VMEM((CH, B), jnp.int32),
        pltpu.VMEM((CH, B), jnp.int32),
        pltpu.VMEM((CH, B), jnp.int32),
        pltpu.VMEM((NBUF, B, D), jnp.float32),
        pltpu.VMEM_SHARED((NPAD, D), jnp.float32),
        pltpu.SemaphoreType.DMA,
        pltpu.SemaphoreType.DMA,
        pltpu.SemaphoreType.DMA,
        pltpu.SemaphoreType.DMA,
        pltpu.SemaphoreType.DMA,
    ]

    def body(src_hbm, dst_hbm, x_hbm, z_hbm, out_hbm,
             srcA, dstA, srcB, dstB, rows_v, acc_sh,
             gsem0, gsem1, ssem0, ssem1, isem):
        gsems = [gsem0, gsem1, ssem0, ssem1]
        ssems = gsems
        c = lax.axis_index("c")
        s = lax.axis_index("s")
        wid = s * NC + c
        row0 = s * ROWS_PER_TILE

        pltpu.sync_copy(z_hbm, rows_v.at[0])
        for i in range(ROWS_PER_TILE // B):
            pltpu.sync_copy(rows_v.at[0], acc_sh.at[pl.ds(row0 + i * B, B)])
        plsc.subcore_barrier()

        def gather(sv, j, buf):
            pltpu.async_copy(x_hbm.at[sv.at[j]], rows_v.at[buf], gsems[buf])

        def gwait(sv, j, buf):
            pltpu.make_async_copy(x_hbm.at[sv.at[j]], rows_v.at[buf],
                                  gsems[buf]).wait()

        def scatter(dv, j, buf):
            pltpu.sync_copy(rows_v.at[buf], acc_sh.at[dv.at[j]], add=True)

        def iwait(dst_ref):
            pltpu.make_async_copy(src_hbm.at[wid, pl.ds(0, CH)],
                                  dst_ref, isem).wait()

        NITER = STEPS // (2 * CH)

        def half(i, sv, dv, first):
            for p in range(CH):
                b = p % NBUF
                gwait(sv, p, b)
                scatter(dv, p, b)
                nxt = p + NBUF
                if nxt < CH:
                    gather(sv, nxt, b)
                elif first:
                    if nxt == CH:
                        iwait(srcB)
                        iwait(dstB)
                    gather(srcB, nxt - CH, b)
                else:
                    @pl.when(i + 1 < NITER)
                    def _():
                        if nxt == CH:
                            iwait(srcA)
                            iwait(dstA)
                        gather(srcA, nxt - CH, b)

        def iteration(i, carry):
            cA = 2 * i
            cB = 2 * i + 1
            half(i, srcA, dstA, True)
            @pl.when(i + 1 < NITER)
            def _():
                pltpu.async_copy(
                    src_hbm.at[wid, pl.ds((cA + 2) * CH, CH)], srcA, isem)
                pltpu.async_copy(
                    dst_hbm.at[wid, pl.ds((cA + 2) * CH, CH)], dstA, isem)
            half(i, srcB, dstB, False)
            @pl.when(i + 1 < NITER)
            def _():
                pltpu.async_copy(
                    src_hbm.at[wid, pl.ds((cB + 2) * CH, CH)], srcB, isem)
                pltpu.async_copy(
                    dst_hbm.at[wid, pl.ds((cB + 2) * CH, CH)], dstB, isem)
            return carry

        pltpu.sync_copy(src_hbm.at[wid, pl.ds(0, CH)], srcA)
        pltpu.sync_copy(dst_hbm.at[wid, pl.ds(0, CH)], dstA)
        pltpu.async_copy(src_hbm.at[wid, pl.ds(CH, CH)], srcB, isem)
        pltpu.async_copy(dst_hbm.at[wid, pl.ds(CH, CH)], dstB, isem)
        for b in range(NBUF):
            gather(srcA, b, b)
        lax.fori_loop(0, NITER, iteration, 0)

        plsc.subcore_barrier()
        out_base = c * NPAD + row0
        for i in range(ROWS_PER_TILE // B):
            pltpu.sync_copy(acc_sh.at[pl.ds(row0 + i * B, B)], rows_v.at[0])
            pltpu.sync_copy(rows_v.at[0],
                            out_hbm.at[pl.ds(out_base + i * B, B)])

    return pl.kernel(
        body, out_type=out, mesh=mesh, scratch_types=scratch,
        compiler_params=pltpu.CompilerParams(
            use_tc_tiling_on_sc=tc_tiling,
            disable_bounds_checks=True,
            disable_semaphore_checks=True))


def _make_deg():
    mesh = plsc.VectorSubcoreMesh(core_axis_name="c", subcore_axis_name="s")
    out = jax.ShapeDtypeStruct((NC * NPAD, 16), jnp.float32)
    scratch = [
        pltpu.VMEM((CH, B), jnp.int32),
        pltpu.VMEM((B, 16), jnp.float32),
        pltpu.VMEM_SHARED((NPAD, 16), jnp.float32),
    ]

    def body(dst_hbm, z16_hbm, ones_hbm, deg_hbm, dst_v, uno_v, deg_sh):
        c = lax.axis_index("c")
        s = lax.axis_index("s")
        wid = s * NC + c
        row0 = s * ROWS_PER_TILE

        pltpu.sync_copy(z16_hbm, uno_v)
        for i in range(ROWS_PER_TILE // B):
            pltpu.sync_copy(uno_v, deg_sh.at[pl.ds(row0 + i * B, B)])
        pltpu.sync_copy(ones_hbm, uno_v)
        plsc.subcore_barrier()

        def chunk_step(k, carry):
            pltpu.sync_copy(dst_hbm.at[wid, pl.ds(k * CH, CH)], dst_v)
            for jj in range(CH):
                pltpu.sync_copy(uno_v, deg_sh.at[dst_v.at[jj]], add=True)
            return carry
        lax.fori_loop(0, STEPS // CH, chunk_step, 0)

        plsc.subcore_barrier()
        out_base = c * NPAD + row0
        for i in range(ROWS_PER_TILE // B):
            pltpu.sync_copy(deg_sh.at[pl.ds(row0 + i * B, B)], uno_v)
            pltpu.sync_copy(uno_v, deg_hbm.at[pl.ds(out_base + i * B, B)])

    return pl.kernel(
        body, out_type=out, mesh=mesh, scratch_types=scratch,
        compiler_params=pltpu.CompilerParams(use_tc_tiling_on_sc=False))


_agg_x = _make_agg(F)
_agg_g = _make_agg(CP, tc_tiling=False)
_deg = _make_deg()


BN = 1024


def _tc1_body(p_ref, d_ref, w_ref, b_ref, o_ref):
    inv = 1.0 / jnp.maximum(d_ref[0, :, 0] + d_ref[1, :, 0], 1.0)
    h = (p_ref[0] + p_ref[1]) * inv[:, None]
    y = jnp.dot(h, w_ref[...], preferred_element_type=jnp.float32,
                precision=lax.Precision.HIGHEST)
    o_ref[...] = jnp.maximum(y + b_ref[...], 0.0)


def _tc2_body(p_ref, d_ref, w1_ref, b1_ref, w2_ref, o_ref):
    inv = 1.0 / jnp.maximum(d_ref[0, :, 0] + d_ref[1, :, 0], 1.0)
    h = (p_ref[0] + p_ref[1]) * inv[:, None]
    y = jnp.dot(h, w1_ref[...], preferred_element_type=jnp.float32,
                precision=lax.Precision.HIGHEST)
    y = jnp.maximum(y + b1_ref[...], 0.0)
    o_ref[...] = jnp.dot(y, w2_ref[...], preferred_element_type=jnp.float32,
                         precision=lax.Precision.HIGHEST)


def _tc3_body(p_ref, d_ref, b_ref, o_ref):
    inv = 1.0 / jnp.maximum(d_ref[0, :, 0] + d_ref[1, :, 0], 1.0)
    z = (p_ref[0] + p_ref[1]) * inv[:, None] + b_ref[...]
    mask = lax.broadcasted_iota(jnp.int32, (BN, CP), 1) < C
    zm = jnp.where(mask, z, -jnp.inf)
    m = jnp.max(zm, axis=1, keepdims=True)
    e = jnp.where(mask, jnp.exp(z - m), 0.0)
    lse = jnp.log(jnp.sum(e, axis=1, keepdims=True))
    o_ref[...] = z - m - lse


def _pspec(d):
    return pl.BlockSpec((2, BN, d), lambda i: (0, i, 0))


def _dspec():
    return pl.BlockSpec((2, BN, 16), lambda i: (0, i, 0))


def _full(shape):
    nd = len(shape)
    return pl.BlockSpec(shape, lambda i: (0,) * nd)


def _tc1(p, d, w, b):
    return pl.pallas_call(
        _tc1_body,
        grid=(NPAD // BN,),
        in_specs=[_pspec(H), _dspec(), _full((F, H)), _full((1, H))],
        out_specs=pl.BlockSpec((BN, H), lambda i: (i, 0)),
        out_shape=jax.ShapeDtypeStruct((NPAD, H), jnp.float32),
    )(p, d, w, b)


def _tc2(p, d, w1, b1, w2):
    return pl.pallas_call(
        _tc2_body,
        grid=(NPAD // BN,),
        in_specs=[_pspec(H), _dspec(), _full((H, H)), _full((1, H)),
                  _full((H, CP))],
        out_specs=pl.BlockSpec((BN, CP), lambda i: (i, 0)),
        out_shape=jax.ShapeDtypeStruct((NPAD, CP), jnp.float32),
    )(p, d, w1, b1, w2)


def _tc3(p, d, b):
    return pl.pallas_call(
        _tc3_body,
        grid=(NPAD // BN,),
        in_specs=[_pspec(CP), _dspec(), _full((1, CP))],
        out_specs=pl.BlockSpec((BN, CP), lambda i: (i, 0)),
        out_shape=jax.ShapeDtypeStruct((NPAD, CP), jnp.float32),
    )(p, d, b)


def kernel(features, edge_index, W0, b0, W1, b1, W2, b2):
    ei = edge_index.astype(jnp.int32)
    pad = NW * EPW - E
    src = jnp.concatenate([ei[0], jnp.full((pad,), N, jnp.int32)])
    dst = jnp.concatenate([ei[1], jnp.full((pad,), N, jnp.int32)])
    src = src.reshape(NW, STEPS, B)
    dst = dst.reshape(NW, STEPS, B)

    x = jnp.zeros((NPAD, F), jnp.float32).at[:N].set(features)
    zF = jnp.zeros((B, F), jnp.float32)
    zCP = jnp.zeros((B, CP), jnp.float32)
    z16 = jnp.zeros((B, 16), jnp.float32)
    ones16 = jnp.ones((B, 16), jnp.float32)

    w2p = jnp.zeros((H, CP), jnp.float32).at[:, :C].set(W2)
    b2p = jnp.zeros((1, CP), jnp.float32).at[0, :C].set(b2)

    deg = _deg(dst, z16, ones16).reshape(NC, NPAD, 16)
    p0 = _agg_x(src, dst, x, zF).reshape(NC, NPAD, F)

    h1 = _tc1(p0, deg, W0, b0.reshape(1, H))
    q = _agg_x(src, dst, h1, zF).reshape(NC, NPAD, H)
    g = _tc2(q, deg, W1, b1.reshape(1, H), w2p)
    r = _agg_g(src, dst, g, zCP).reshape(NC, NPAD, CP)
    z = _tc3(r, deg, b2p)
    return z[:N, :C]


# speedup vs baseline: 1.6665x; 1.3090x over previous
import jax
import jax.numpy as jnp
from jax import lax
from jax.experimental import pallas as pl
from jax.experimental.pallas import tpu as pltpu
from jax.experimental.pallas import tpu_sc as plsc

N = 10000
E = 320000
F = 128
H = 128
C = 40
CP = 48
NPAD = 10240
NC = 2
NS = 16
NW = NC * NS
B = 128
STEPS = 80
CH = 8
NBUF = 2
EPW = STEPS * B
ROWS_PER_TILE = NPAD // NS


def _make_agg(D, tc_tiling=True, dt=jnp.float32):
    mesh = plsc.VectorSubcoreMesh(core_axis_name="c", subcore_axis_name="s")
    out = jax.ShapeDtypeStruct((NC * NPAD, D), dt)
    scratch = [
        pltpu.VMEM((CH, B), jnp.int32),
        pltpu.VMEM((CH, B), jnp.int32),
        pltpu.VMEM((CH, B), jnp.int32),
        pltpu.VMEM((CH, B), jnp.int32),
        pltpu.VMEM((NBUF, B, D), dt),
        pltpu.VMEM_SHARED((NPAD, D), dt),
        pltpu.SemaphoreType.DMA,
        pltpu.SemaphoreType.DMA,
        pltpu.SemaphoreType.DMA,
        pltpu.SemaphoreType.DMA,
        pltpu.SemaphoreType.DMA,
    ]

    def body(src_hbm, dst_hbm, x_hbm, z_hbm, out_hbm,
             srcA, dstA, srcB, dstB, rows_v, acc_sh,
             gsem0, gsem1, ssem0, ssem1, isem):
        gsems = [gsem0, gsem1, ssem0, ssem1]
        ssems = gsems
        c = lax.axis_index("c")
        s = lax.axis_index("s")
        wid = s * NC + c
        row0 = s * ROWS_PER_TILE

        pltpu.sync_copy(z_hbm, rows_v.at[0])
        for i in range(ROWS_PER_TILE // B):
            pltpu.sync_copy(rows_v.at[0], acc_sh.at[pl.ds(row0 + i * B, B)])
        plsc.subcore_barrier()

        def gather(sv, j, buf):
            pltpu.async_copy(x_hbm.at[sv.at[j]], rows_v.at[buf], gsems[buf])

        def gwait(sv, j, buf):
            pltpu.make_async_copy(x_hbm.at[sv.at[j]], rows_v.at[buf],
                                  gsems[buf]).wait()

        def scatter(dv, j, buf):
            pltpu.sync_copy(rows_v.at[buf], acc_sh.at[dv.at[j]], add=True)

        def iwait(dst_ref):
            pltpu.make_async_copy(src_hbm.at[wid, pl.ds(0, CH)],
                                  dst_ref, isem).wait()

        NITER = STEPS // (2 * CH)

        def half(i, sv, dv, first):
            for p in range(CH):
                b = p % NBUF
                gwait(sv, p, b)
                scatter(dv, p, b)
                nxt = p + NBUF
                if nxt < CH:
                    gather(sv, nxt, b)
                elif first:
                    if nxt == CH:
                        iwait(srcB)
                        iwait(dstB)
                    gather(srcB, nxt - CH, b)
                else:
                    @pl.when(i + 1 < NITER)
                    def _():
                        if nxt == CH:
                            iwait(srcA)
                            iwait(dstA)
                        gather(srcA, nxt - CH, b)

        def iteration(i, carry):
            cA = 2 * i
            cB = 2 * i + 1
            half(i, srcA, dstA, True)
            @pl.when(i + 1 < NITER)
            def _():
                pltpu.async_copy(
                    src_hbm.at[wid, pl.ds((cA + 2) * CH, CH)], srcA, isem)
                pltpu.async_copy(
                    dst_hbm.at[wid, pl.ds((cA + 2) * CH, CH)], dstA, isem)
            half(i, srcB, dstB, False)
            @pl.when(i + 1 < NITER)
            def _():
                pltpu.async_copy(
                    src_hbm.at[wid, pl.ds((cB + 2) * CH, CH)], srcB, isem)
                pltpu.async_copy(
                    dst_hbm.at[wid, pl.ds((cB + 2) * CH, CH)], dstB, isem)
            return carry

        pltpu.sync_copy(src_hbm.at[wid, pl.ds(0, CH)], srcA)
        pltpu.sync_copy(dst_hbm.at[wid, pl.ds(0, CH)], dstA)
        pltpu.async_copy(src_hbm.at[wid, pl.ds(CH, CH)], srcB, isem)
        pltpu.async_copy(dst_hbm.at[wid, pl.ds(CH, CH)], dstB, isem)
        for b in range(NBUF):
            gather(srcA, b, b)
        lax.fori_loop(0, NITER, iteration, 0)

        plsc.subcore_barrier()
        out_base = c * NPAD + row0
        for i in range(ROWS_PER_TILE // B):
            pltpu.sync_copy(acc_sh.at[pl.ds(row0 + i * B, B)], rows_v.at[0])
            pltpu.sync_copy(rows_v.at[0],
                            out_hbm.at[pl.ds(out_base + i * B, B)])

    return pl.kernel(
        body, out_type=out, mesh=mesh, scratch_types=scratch,
        compiler_params=pltpu.CompilerParams(
            use_tc_tiling_on_sc=tc_tiling,
            disable_bounds_checks=True,
            disable_semaphore_checks=True))


def _make_deg():
    mesh = plsc.VectorSubcoreMesh(core_axis_name="c", subcore_axis_name="s")
    out = jax.ShapeDtypeStruct((NC * NPAD, 16), jnp.float32)
    scratch = [
        pltpu.VMEM((CH, B), jnp.int32),
        pltpu.VMEM((B, 16), jnp.float32),
        pltpu.VMEM_SHARED((NPAD, 16), jnp.float32),
    ]

    def body(dst_hbm, z16_hbm, ones_hbm, deg_hbm, dst_v, uno_v, deg_sh):
        c = lax.axis_index("c")
        s = lax.axis_index("s")
        wid = s * NC + c
        row0 = s * ROWS_PER_TILE

        pltpu.sync_copy(z16_hbm, uno_v)
        for i in range(ROWS_PER_TILE // B):
            pltpu.sync_copy(uno_v, deg_sh.at[pl.ds(row0 + i * B, B)])
        pltpu.sync_copy(ones_hbm, uno_v)
        plsc.subcore_barrier()

        def chunk_step(k, carry):
            pltpu.sync_copy(dst_hbm.at[wid, pl.ds(k * CH, CH)], dst_v)
            for jj in range(CH):
                pltpu.sync_copy(uno_v, deg_sh.at[dst_v.at[jj]], add=True)
            return carry
        lax.fori_loop(0, STEPS // CH, chunk_step, 0)

        plsc.subcore_barrier()
        out_base = c * NPAD + row0
        for i in range(ROWS_PER_TILE // B):
            pltpu.sync_copy(deg_sh.at[pl.ds(row0 + i * B, B)], uno_v)
            pltpu.sync_copy(uno_v, deg_hbm.at[pl.ds(out_base + i * B, B)])

    return pl.kernel(
        body, out_type=out, mesh=mesh, scratch_types=scratch,
        compiler_params=pltpu.CompilerParams(use_tc_tiling_on_sc=False))


_agg_x = _make_agg(F, tc_tiling=False, dt=jnp.bfloat16)
_agg_g = _make_agg(CP, tc_tiling=False, dt=jnp.bfloat16)
_deg = _make_deg()


BN = 1024


def _tc1_body(p_ref, d_ref, w_ref, b_ref, o_ref):
    inv = 1.0 / jnp.maximum(d_ref[0, :, 0] + d_ref[1, :, 0], 1.0)
    psum = p_ref[0].astype(jnp.float32) + p_ref[1].astype(jnp.float32)
    h = psum * inv[:, None]
    y = jnp.dot(h, w_ref[...], preferred_element_type=jnp.float32,
                precision=lax.Precision.HIGHEST)
    o_ref[...] = jnp.maximum(y + b_ref[...], 0.0).astype(jnp.bfloat16)


def _tc2_body(p_ref, d_ref, w1_ref, b1_ref, w2_ref, o_ref):
    inv = 1.0 / jnp.maximum(d_ref[0, :, 0] + d_ref[1, :, 0], 1.0)
    psum = p_ref[0].astype(jnp.float32) + p_ref[1].astype(jnp.float32)
    h = psum * inv[:, None]
    y = jnp.dot(h, w1_ref[...], preferred_element_type=jnp.float32,
                precision=lax.Precision.HIGHEST)
    y = jnp.maximum(y + b1_ref[...], 0.0)
    g = jnp.dot(y, w2_ref[...], preferred_element_type=jnp.float32,
                precision=lax.Precision.HIGHEST)
    o_ref[...] = g.astype(jnp.bfloat16)


def _tc3_body(p_ref, d_ref, b_ref, o_ref):
    inv = 1.0 / jnp.maximum(d_ref[0, :, 0] + d_ref[1, :, 0], 1.0)
    psum = p_ref[0].astype(jnp.float32) + p_ref[1].astype(jnp.float32)
    z = psum * inv[:, None] + b_ref[...]
    mask = lax.broadcasted_iota(jnp.int32, (BN, CP), 1) < C
    zm = jnp.where(mask, z, -jnp.inf)
    m = jnp.max(zm, axis=1, keepdims=True)
    e = jnp.where(mask, jnp.exp(z - m), 0.0)
    lse = jnp.log(jnp.sum(e, axis=1, keepdims=True))
    o_ref[...] = z - m - lse


def _pspec(d):
    return pl.BlockSpec((2, BN, d), lambda i: (0, i, 0))


def _dspec():
    return pl.BlockSpec((2, BN, 16), lambda i: (0, i, 0))


def _full(shape):
    nd = len(shape)
    return pl.BlockSpec(shape, lambda i: (0,) * nd)


def _tc1(p, d, w, b):
    return pl.pallas_call(
        _tc1_body,
        grid=(NPAD // BN,),
        in_specs=[_pspec(H), _dspec(), _full((F, H)), _full((1, H))],
        out_specs=pl.BlockSpec((BN, H), lambda i: (i, 0)),
        out_shape=jax.ShapeDtypeStruct((NPAD, H), jnp.bfloat16),
    )(p, d, w, b)


def _tc2(p, d, w1, b1, w2):
    return pl.pallas_call(
        _tc2_body,
        grid=(NPAD // BN,),
        in_specs=[_pspec(H), _dspec(), _full((H, H)), _full((1, H)),
                  _full((H, CP))],
        out_specs=pl.BlockSpec((BN, CP), lambda i: (i, 0)),
        out_shape=jax.ShapeDtypeStruct((NPAD, CP), jnp.bfloat16),
    )(p, d, w1, b1, w2)


def _tc3(p, d, b):
    return pl.pallas_call(
        _tc3_body,
        grid=(NPAD // BN,),
        in_specs=[_pspec(CP), _dspec(), _full((1, CP))],
        out_specs=pl.BlockSpec((BN, CP), lambda i: (i, 0)),
        out_shape=jax.ShapeDtypeStruct((NPAD, CP), jnp.float32),
    )(p, d, b)


def kernel(features, edge_index, W0, b0, W1, b1, W2, b2):
    ei = edge_index.astype(jnp.int32)
    pad = NW * EPW - E
    src = jnp.concatenate([ei[0], jnp.full((pad,), N, jnp.int32)])
    dst = jnp.concatenate([ei[1], jnp.full((pad,), N, jnp.int32)])
    src = src.reshape(NW, STEPS, B)
    dst = dst.reshape(NW, STEPS, B)

    x = jnp.zeros((NPAD, F), jnp.bfloat16).at[:N].set(
        features.astype(jnp.bfloat16))
    zF = jnp.zeros((B, F), jnp.bfloat16)
    zCP = jnp.zeros((B, CP), jnp.bfloat16)
    z16 = jnp.zeros((B, 16), jnp.float32)
    ones16 = jnp.ones((B, 16), jnp.float32)

    w2p = jnp.zeros((H, CP), jnp.float32).at[:, :C].set(W2)
    b2p = jnp.zeros((1, CP), jnp.float32).at[0, :C].set(b2)

    deg = _deg(dst, z16, ones16).reshape(NC, NPAD, 16)
    p0 = _agg_x(src, dst, x, zF).reshape(NC, NPAD, F)

    h1 = _tc1(p0, deg, W0, b0.reshape(1, H))
    q = _agg_x(src, dst, h1, zF).reshape(NC, NPAD, H)
    g = _tc2(q, deg, W1, b1.reshape(1, H), w2p)
    r = _agg_g(src, dst, g, zCP).reshape(NC, NPAD, CP)
    z = _tc3(r, deg, b2p)
    return z[:N, :C]


# speedup vs baseline: 2.1799x; 1.3081x over previous
import jax
import jax.numpy as jnp
from jax import lax
from jax.experimental import pallas as pl
from jax.experimental.pallas import tpu as pltpu
from jax.experimental.pallas import tpu_sc as plsc

N = 10000
E = 320000
F = 128
H = 128
C = 40
CP = 48
NPAD = 10240
NC = 2
NS = 16
NW = NC * NS
B = 128
STEPS = 80
CH = 8
NBUF = 4
EPW = STEPS * B
ROWS_PER_TILE = NPAD // NS


def _make_agg(D, tc_tiling=True, dt=jnp.float32, with_deg=False):
    mesh = plsc.VectorSubcoreMesh(core_axis_name="c", subcore_axis_name="s")
    outs = [jax.ShapeDtypeStruct((NC * NPAD, D), dt)]
    scratch = [
        pltpu.VMEM((CH, B), jnp.int32),
        pltpu.VMEM((CH, B), jnp.int32),
        pltpu.VMEM((CH, B), jnp.int32),
        pltpu.VMEM((CH, B), jnp.int32),
        pltpu.VMEM((NBUF, B, D), dt),
        pltpu.VMEM_SHARED((NPAD, D), dt),
        pltpu.SemaphoreType.DMA,
        pltpu.SemaphoreType.DMA,
        pltpu.SemaphoreType.DMA,
        pltpu.SemaphoreType.DMA,
        pltpu.SemaphoreType.DMA,
    ]
    if with_deg:
        outs.append(jax.ShapeDtypeStruct((NC * NPAD, 16), jnp.float32))
        scratch += [
            pltpu.VMEM((B, 16), jnp.float32),
            pltpu.VMEM_SHARED((NPAD, 16), jnp.float32),
        ]

    def body(src_hbm, dst_hbm, x_hbm, z_hbm, *rest):
        if with_deg:
            (z16_hbm, ones_hbm, out_hbm, deg_hbm, srcA, dstA, srcB, dstB,
             rows_v, acc_sh, gsem0, gsem1, ssem0, ssem1, isem,
             uno_v, deg_sh) = rest
        else:
            (out_hbm, srcA, dstA, srcB, dstB, rows_v, acc_sh,
             gsem0, gsem1, ssem0, ssem1, isem) = rest
        gsems = [gsem0, gsem1, ssem0, ssem1]
        c = lax.axis_index("c")
        s = lax.axis_index("s")
        wid = s * NC + c
        row0 = s * ROWS_PER_TILE

        pltpu.sync_copy(z_hbm, rows_v.at[0])
        if with_deg:
            pltpu.sync_copy(z16_hbm, uno_v)
        for i in range(ROWS_PER_TILE // B):
            pltpu.sync_copy(rows_v.at[0], acc_sh.at[pl.ds(row0 + i * B, B)])
            if with_deg:
                pltpu.sync_copy(uno_v, deg_sh.at[pl.ds(row0 + i * B, B)])
        if with_deg:
            pltpu.sync_copy(ones_hbm, uno_v)
        plsc.subcore_barrier()

        def gather(sv, j, buf):
            pltpu.async_copy(x_hbm.at[sv.at[j]], rows_v.at[buf], gsems[buf])

        def gwait(sv, j, buf):
            pltpu.make_async_copy(x_hbm.at[sv.at[j]], rows_v.at[buf],
                                  gsems[buf]).wait()

        def scatter(dv, j, buf):
            pltpu.sync_copy(rows_v.at[buf], acc_sh.at[dv.at[j]], add=True)

        def iwait(dst_ref):
            pltpu.make_async_copy(src_hbm.at[wid, pl.ds(0, CH)],
                                  dst_ref, isem).wait()

        NITER = STEPS // (2 * CH)

        def half(i, sv, dv, first):
            for p in range(CH):
                b = p % NBUF
                gwait(sv, p, b)
                scatter(dv, p, b)
                if with_deg:
                    pltpu.sync_copy(uno_v, deg_sh.at[dv.at[p]], add=True)
                nxt = p + NBUF
                if nxt < CH:
                    gather(sv, nxt, b)
                elif first:
                    if nxt == CH:
                        iwait(srcB)
                        iwait(dstB)
                    gather(srcB, nxt - CH, b)
                else:
                    @pl.when(i + 1 < NITER)
                    def _():
                        if nxt == CH:
                            iwait(srcA)
                            iwait(dstA)
                        gather(srcA, nxt - CH, b)

        def iteration(i, carry):
            cA = 2 * i
            cB = 2 * i + 1
            half(i, srcA, dstA, True)
            @pl.when(i + 1 < NITER)
            def _():
                pltpu.async_copy(
                    src_hbm.at[wid, pl.ds((cA + 2) * CH, CH)], srcA, isem)
                pltpu.async_copy(
                    dst_hbm.at[wid, pl.ds((cA + 2) * CH, CH)], dstA, isem)
            half(i, srcB, dstB, False)
            @pl.when(i + 1 < NITER)
            def _():
                pltpu.async_copy(
                    src_hbm.at[wid, pl.ds((cB + 2) * CH, CH)], srcB, isem)
                pltpu.async_copy(
                    dst_hbm.at[wid, pl.ds((cB + 2) * CH, CH)], dstB, isem)
            return carry

        pltpu.sync_copy(src_hbm.at[wid, pl.ds(0, CH)], srcA)
        pltpu.sync_copy(dst_hbm.at[wid, pl.ds(0, CH)], dstA)
        pltpu.async_copy(src_hbm.at[wid, pl.ds(CH, CH)], srcB, isem)
        pltpu.async_copy(dst_hbm.at[wid, pl.ds(CH, CH)], dstB, isem)
        for b in range(NBUF):
            gather(srcA, b, b)
        lax.fori_loop(0, NITER, iteration, 0)

        plsc.subcore_barrier()
        out_base = c * NPAD + row0
        for i in range(ROWS_PER_TILE // B):
            pltpu.sync_copy(acc_sh.at[pl.ds(row0 + i * B, B)], rows_v.at[0])
            pltpu.sync_copy(rows_v.at[0],
                            out_hbm.at[pl.ds(out_base + i * B, B)])
            if with_deg:
                pltpu.sync_copy(deg_sh.at[pl.ds(row0 + i * B, B)], uno_v)
                pltpu.sync_copy(uno_v,
                                deg_hbm.at[pl.ds(out_base + i * B, B)])

    return pl.kernel(
        body, out_type=tuple(outs) if with_deg else outs[0],
        mesh=mesh, scratch_types=scratch,
        compiler_params=pltpu.CompilerParams(
            use_tc_tiling_on_sc=tc_tiling,
            disable_bounds_checks=True,
            disable_semaphore_checks=True))


_agg_xd = _make_agg(F, tc_tiling=False, dt=jnp.bfloat16, with_deg=True)
_agg_x = _make_agg(F, tc_tiling=False, dt=jnp.bfloat16)
_agg_g = _make_agg(CP, tc_tiling=False, dt=jnp.bfloat16)


BN = 1024


def _tc1_body(p_ref, d_ref, w_ref, b_ref, o_ref):
    inv = 1.0 / jnp.maximum(d_ref[0, :, 0] + d_ref[1, :, 0], 1.0)
    psum = p_ref[0].astype(jnp.float32) + p_ref[1].astype(jnp.float32)
    h = psum * inv[:, None]
    y = jnp.dot(h, w_ref[...], preferred_element_type=jnp.float32,
                precision=lax.Precision.HIGHEST)
    o_ref[...] = jnp.maximum(y + b_ref[...], 0.0).astype(jnp.bfloat16)


def _tc2_body(p_ref, d_ref, w1_ref, b1_ref, w2_ref, o_ref):
    inv = 1.0 / jnp.maximum(d_ref[0, :, 0] + d_ref[1, :, 0], 1.0)
    psum = p_ref[0].astype(jnp.float32) + p_ref[1].astype(jnp.float32)
    h = psum * inv[:, None]
    y = jnp.dot(h, w1_ref[...], preferred_element_type=jnp.float32,
                precision=lax.Precision.HIGHEST)
    y = jnp.maximum(y + b1_ref[...], 0.0)
    g = jnp.dot(y, w2_ref[...], preferred_element_type=jnp.float32,
                precision=lax.Precision.HIGHEST)
    o_ref[...] = g.astype(jnp.bfloat16)


def _tc3_body(p_ref, d_ref, b_ref, o_ref):
    inv = 1.0 / jnp.maximum(d_ref[0, :, 0] + d_ref[1, :, 0], 1.0)
    psum = p_ref[0].astype(jnp.float32) + p_ref[1].astype(jnp.float32)
    z = psum * inv[:, None] + b_ref[...]
    mask = lax.broadcasted_iota(jnp.int32, (BN, CP), 1) < C
    zm = jnp.where(mask, z, -jnp.inf)
    m = jnp.max(zm, axis=1, keepdims=True)
    e = jnp.where(mask, jnp.exp(z - m), 0.0)
    lse = jnp.log(jnp.sum(e, axis=1, keepdims=True))
    o_ref[...] = z - m - lse


def _pspec(d):
    return pl.BlockSpec((2, BN, d), lambda i: (0, i, 0))


def _dspec():
    return pl.BlockSpec((2, BN, 16), lambda i: (0, i, 0))


def _full(shape):
    nd = len(shape)
    return pl.BlockSpec(shape, lambda i: (0,) * nd)


def _tc1(p, d, w, b):
    return pl.pallas_call(
        _tc1_body,
        grid=(NPAD // BN,),
        in_specs=[_pspec(H), _dspec(), _full((F, H)), _full((1, H))],
        out_specs=pl.BlockSpec((BN, H), lambda i: (i, 0)),
        out_shape=jax.ShapeDtypeStruct((NPAD, H), jnp.bfloat16),
    )(p, d, w, b)


def _tc2(p, d, w1, b1, w2):
    return pl.pallas_call(
        _tc2_body,
        grid=(NPAD // BN,),
        in_specs=[_pspec(H), _dspec(), _full((H, H)), _full((1, H)),
                  _full((H, CP))],
        out_specs=pl.BlockSpec((BN, CP), lambda i: (i, 0)),
        out_shape=jax.ShapeDtypeStruct((NPAD, CP), jnp.bfloat16),
    )(p, d, w1, b1, w2)


def _tc3(p, d, b):
    return pl.pallas_call(
        _tc3_body,
        grid=(NPAD // BN,),
        in_specs=[_pspec(CP), _dspec(), _full((1, CP))],
        out_specs=pl.BlockSpec((BN, CP), lambda i: (i, 0)),
        out_shape=jax.ShapeDtypeStruct((NPAD, CP), jnp.float32),
    )(p, d, b)


def kernel(features, edge_index, W0, b0, W1, b1, W2, b2):
    ei = edge_index.astype(jnp.int32)
    pad = NW * EPW - E
    src = jnp.concatenate([ei[0], jnp.full((pad,), N, jnp.int32)])
    dst = jnp.concatenate([ei[1], jnp.full((pad,), N, jnp.int32)])
    src = src.reshape(NW, STEPS, B)
    dst = dst.reshape(NW, STEPS, B)

    x = jnp.zeros((NPAD, F), jnp.bfloat16).at[:N].set(
        features.astype(jnp.bfloat16))
    zF = jnp.zeros((B, F), jnp.bfloat16)
    zCP = jnp.zeros((B, CP), jnp.bfloat16)
    z16 = jnp.zeros((B, 16), jnp.float32)
    ones16 = jnp.ones((B, 16), jnp.float32)

    w2p = jnp.zeros((H, CP), jnp.float32).at[:, :C].set(W2)
    b2p = jnp.zeros((1, CP), jnp.float32).at[0, :C].set(b2)

    p0, deg = _agg_xd(src, dst, x, zF, z16, ones16)
    p0 = p0.reshape(NC, NPAD, F)
    deg = deg.reshape(NC, NPAD, 16)

    h1 = _tc1(p0, deg, W0, b0.reshape(1, H))
    q = _agg_x(src, dst, h1, zF).reshape(NC, NPAD, H)
    g = _tc2(q, deg, W1, b1.reshape(1, H), w2p)
    r = _agg_g(src, dst, g, zCP).reshape(NC, NPAD, CP)
    z = _tc3(r, deg, b2p)
    return z[:N, :C]


# speedup vs baseline: 2.1912x; 1.0052x over previous
import jax
import jax.numpy as jnp
from jax import lax
from jax.experimental import pallas as pl
from jax.experimental.pallas import tpu as pltpu
from jax.experimental.pallas import tpu_sc as plsc

N = 10000
E = 320000
F = 128
H = 128
C = 40
CP = 48
NPAD = 10240
NC = 2
NS = 16
NW = NC * NS
B = 128
STEPS = 80
CH = 8
NBUF = 8
EPW = STEPS * B
ROWS_PER_TILE = NPAD // NS


def _make_agg(D, tc_tiling=True, dt=jnp.float32, with_deg=False):
    mesh = plsc.VectorSubcoreMesh(core_axis_name="c", subcore_axis_name="s")
    outs = [jax.ShapeDtypeStruct((NC * NPAD, D), dt)]
    scratch = [
        pltpu.VMEM((CH, B), jnp.int32),
        pltpu.VMEM((CH, B), jnp.int32),
        pltpu.VMEM((CH, B), jnp.int32),
        pltpu.VMEM((CH, B), jnp.int32),
        pltpu.VMEM((NBUF, B, D), dt),
        pltpu.VMEM_SHARED((NPAD, D), dt),
    ] + [pltpu.SemaphoreType.DMA] * NBUF + [
        pltpu.SemaphoreType.DMA,
    ]
    if with_deg:
        outs.append(jax.ShapeDtypeStruct((NC * NPAD, 16), jnp.float32))
        scratch += [
            pltpu.VMEM((B, 16), jnp.float32),
            pltpu.VMEM_SHARED((NPAD, 16), jnp.float32),
        ]

    def body(src_hbm, dst_hbm, x_hbm, z_hbm, *rest):
        if with_deg:
            (z16_hbm, ones_hbm, out_hbm, deg_hbm, srcA, dstA, srcB, dstB,
             rows_v, acc_sh, *sems, uno_v, deg_sh) = rest
        else:
            (out_hbm, srcA, dstA, srcB, dstB, rows_v, acc_sh, *sems) = rest
        gsems = sems[:NBUF]
        isem = sems[NBUF]
        c = lax.axis_index("c")
        s = lax.axis_index("s")
        wid = s * NC + c
        row0 = s * ROWS_PER_TILE

        pltpu.sync_copy(z_hbm, rows_v.at[0])
        if with_deg:
            pltpu.sync_copy(z16_hbm, uno_v)
        for i in range(ROWS_PER_TILE // B):
            pltpu.sync_copy(rows_v.at[0], acc_sh.at[pl.ds(row0 + i * B, B)])
            if with_deg:
                pltpu.sync_copy(uno_v, deg_sh.at[pl.ds(row0 + i * B, B)])
        if with_deg:
            pltpu.sync_copy(ones_hbm, uno_v)
        plsc.subcore_barrier()

        def gather(sv, j, buf):
            pltpu.async_copy(x_hbm.at[sv.at[j]], rows_v.at[buf], gsems[buf])

        def gwait(sv, j, buf):
            pltpu.make_async_copy(x_hbm.at[sv.at[j]], rows_v.at[buf],
                                  gsems[buf]).wait()

        def scatter(dv, j, buf):
            pltpu.sync_copy(rows_v.at[buf], acc_sh.at[dv.at[j]], add=True)

        def iwait(dst_ref):
            pltpu.make_async_copy(src_hbm.at[wid, pl.ds(0, CH)],
                                  dst_ref, isem).wait()

        NITER = STEPS // (2 * CH)

        def half(i, sv, dv, first):
            for p in range(CH):
                b = p % NBUF
                gwait(sv, p, b)
                scatter(dv, p, b)
                if with_deg:
                    pltpu.sync_copy(uno_v, deg_sh.at[dv.at[p]], add=True)
                nxt = p + NBUF
                if nxt < CH:
                    gather(sv, nxt, b)
                elif first:
                    if nxt == CH:
                        iwait(srcB)
                        iwait(dstB)
                    gather(srcB, nxt - CH, b)
                else:
                    @pl.when(i + 1 < NITER)
                    def _():
                        if nxt == CH:
                            iwait(srcA)
                            iwait(dstA)
                        gather(srcA, nxt - CH, b)

        def iteration(i, carry):
            cA = 2 * i
            cB = 2 * i + 1
            half(i, srcA, dstA, True)
            @pl.when(i + 1 < NITER)
            def _():
                pltpu.async_copy(
                    src_hbm.at[wid, pl.ds((cA + 2) * CH, CH)], srcA, isem)
                pltpu.async_copy(
                    dst_hbm.at[wid, pl.ds((cA + 2) * CH, CH)], dstA, isem)
            half(i, srcB, dstB, False)
            @pl.when(i + 1 < NITER)
            def _():
                pltpu.async_copy(
                    src_hbm.at[wid, pl.ds((cB + 2) * CH, CH)], srcB, isem)
                pltpu.async_copy(
                    dst_hbm.at[wid, pl.ds((cB + 2) * CH, CH)], dstB, isem)
            return carry

        pltpu.sync_copy(src_hbm.at[wid, pl.ds(0, CH)], srcA)
        pltpu.sync_copy(dst_hbm.at[wid, pl.ds(0, CH)], dstA)
        pltpu.async_copy(src_hbm.at[wid, pl.ds(CH, CH)], srcB, isem)
        pltpu.async_copy(dst_hbm.at[wid, pl.ds(CH, CH)], dstB, isem)
        for b in range(NBUF):
            gather(srcA, b, b)
        lax.fori_loop(0, NITER, iteration, 0)

        plsc.subcore_barrier()
        out_base = c * NPAD + row0
        for i in range(ROWS_PER_TILE // B):
            pltpu.sync_copy(acc_sh.at[pl.ds(row0 + i * B, B)], rows_v.at[0])
            pltpu.sync_copy(rows_v.at[0],
                            out_hbm.at[pl.ds(out_base + i * B, B)])
            if with_deg:
                pltpu.sync_copy(deg_sh.at[pl.ds(row0 + i * B, B)], uno_v)
                pltpu.sync_copy(uno_v,
                                deg_hbm.at[pl.ds(out_base + i * B, B)])

    return pl.kernel(
        body, out_type=tuple(outs) if with_deg else outs[0],
        mesh=mesh, scratch_types=scratch,
        compiler_params=pltpu.CompilerParams(
            use_tc_tiling_on_sc=tc_tiling,
            disable_bounds_checks=True,
            disable_semaphore_checks=True))


_agg_xd = _make_agg(F, tc_tiling=False, dt=jnp.bfloat16, with_deg=True)
_agg_x = _make_agg(F, tc_tiling=False, dt=jnp.bfloat16)
_agg_g = _make_agg(CP, tc_tiling=False, dt=jnp.bfloat16)


BN = 1024


def _tc1_body(p_ref, d_ref, w_ref, b_ref, o_ref):
    inv = 1.0 / jnp.maximum(d_ref[0, :, 0] + d_ref[1, :, 0], 1.0)
    psum = p_ref[0].astype(jnp.float32) + p_ref[1].astype(jnp.float32)
    h = psum * inv[:, None]
    y = jnp.dot(h, w_ref[...], preferred_element_type=jnp.float32,
                precision=lax.Precision.HIGHEST)
    o_ref[...] = jnp.maximum(y + b_ref[...], 0.0).astype(jnp.bfloat16)


def _tc2_body(p_ref, d_ref, w1_ref, b1_ref, w2_ref, o_ref):
    inv = 1.0 / jnp.maximum(d_ref[0, :, 0] + d_ref[1, :, 0], 1.0)
    psum = p_ref[0].astype(jnp.float32) + p_ref[1].astype(jnp.float32)
    h = psum * inv[:, None]
    y = jnp.dot(h, w1_ref[...], preferred_element_type=jnp.float32,
                precision=lax.Precision.HIGHEST)
    y = jnp.maximum(y + b1_ref[...], 0.0)
    g = jnp.dot(y, w2_ref[...], preferred_element_type=jnp.float32,
                precision=lax.Precision.HIGHEST)
    o_ref[...] = g.astype(jnp.bfloat16)


def _tc3_body(p_ref, d_ref, b_ref, o_ref):
    inv = 1.0 / jnp.maximum(d_ref[0, :, 0] + d_ref[1, :, 0], 1.0)
    psum = p_ref[0].astype(jnp.float32) + p_ref[1].astype(jnp.float32)
    z = psum * inv[:, None] + b_ref[...]
    mask = lax.broadcasted_iota(jnp.int32, (BN, CP), 1) < C
    zm = jnp.where(mask, z, -jnp.inf)
    m = jnp.max(zm, axis=1, keepdims=True)
    e = jnp.where(mask, jnp.exp(z - m), 0.0)
    lse = jnp.log(jnp.sum(e, axis=1, keepdims=True))
    o_ref[...] = z - m - lse


def _pspec(d):
    return pl.BlockSpec((2, BN, d), lambda i: (0, i, 0))


def _dspec():
    return pl.BlockSpec((2, BN, 16), lambda i: (0, i, 0))


def _full(shape):
    nd = len(shape)
    return pl.BlockSpec(shape, lambda i: (0,) * nd)


def _tc1(p, d, w, b):
    return pl.pallas_call(
        _tc1_body,
        grid=(NPAD // BN,),
        in_specs=[_pspec(H), _dspec(), _full((F, H)), _full((1, H))],
        out_specs=pl.BlockSpec((BN, H), lambda i: (i, 0)),
        out_shape=jax.ShapeDtypeStruct((NPAD, H), jnp.bfloat16),
    )(p, d, w, b)


def _tc2(p, d, w1, b1, w2):
    return pl.pallas_call(
        _tc2_body,
        grid=(NPAD // BN,),
        in_specs=[_pspec(H), _dspec(), _full((H, H)), _full((1, H)),
                  _full((H, CP))],
        out_specs=pl.BlockSpec((BN, CP), lambda i: (i, 0)),
        out_shape=jax.ShapeDtypeStruct((NPAD, CP), jnp.bfloat16),
    )(p, d, w1, b1, w2)


def _tc3(p, d, b):
    return pl.pallas_call(
        _tc3_body,
        grid=(NPAD // BN,),
        in_specs=[_pspec(CP), _dspec(), _full((1, CP))],
        out_specs=pl.BlockSpec((BN, CP), lambda i: (i, 0)),
        out_shape=jax.ShapeDtypeStruct((NPAD, CP), jnp.float32),
    )(p, d, b)


def kernel(features, edge_index, W0, b0, W1, b1, W2, b2):
    ei = edge_index.astype(jnp.int32)
    pad = NW * EPW - E
    src = jnp.concatenate([ei[0], jnp.full((pad,), N, jnp.int32)])
    dst = jnp.concatenate([ei[1], jnp.full((pad,), N, jnp.int32)])
    src = src.reshape(NW, STEPS, B)
    dst = dst.reshape(NW, STEPS, B)

    x = jnp.zeros((NPAD, F), jnp.bfloat16).at[:N].set(
        features.astype(jnp.bfloat16))
    zF = jnp.zeros((B, F), jnp.bfloat16)
    zCP = jnp.zeros((B, CP), jnp.bfloat16)
    z16 = jnp.zeros((B, 16), jnp.float32)
    ones16 = jnp.ones((B, 16), jnp.float32)

    w2p = jnp.zeros((H, CP), jnp.float32).at[:, :C].set(W2)
    b2p = jnp.zeros((1, CP), jnp.float32).at[0, :C].set(b2)

    p0, deg = _agg_xd(src, dst, x, zF, z16, ones16)
    p0 = p0.reshape(NC, NPAD, F)
    deg = deg.reshape(NC, NPAD, 16)

    h1 = _tc1(p0, deg, W0, b0.reshape(1, H))
    q = _agg_x(src, dst, h1, zF).reshape(NC, NPAD, H)
    g = _tc2(q, deg, W1, b1.reshape(1, H), w2p)
    r = _agg_g(src, dst, g, zCP).reshape(NC, NPAD, CP)
    z = _tc3(r, deg, b2p)
    return z[:N, :C]


# speedup vs baseline: 2.1975x; 1.0029x over previous
import jax
import jax.numpy as jnp
from jax import lax
from jax.experimental import pallas as pl
from jax.experimental.pallas import tpu as pltpu
from jax.experimental.pallas import tpu_sc as plsc

N = 10000
E = 320000
F = 128
H = 128
C = 40
CP = 48
NPAD = 10240
NC = 2
NS = 16
NW = NC * NS
B = 128
STEPS = 80
CH = 8
NBUF = 8
EPW = STEPS * B
ROWS_PER_TILE = NPAD // NS


def _make_agg(D, tc_tiling=True, dt=jnp.float32, with_deg=False):
    mesh = plsc.VectorSubcoreMesh(core_axis_name="c", subcore_axis_name="s")
    outs = [jax.ShapeDtypeStruct((NC * NPAD, D), dt)]
    scratch = [
        pltpu.VMEM((CH, B), jnp.int32),
        pltpu.VMEM((CH, B), jnp.int32),
        pltpu.VMEM((CH, B), jnp.int32),
        pltpu.VMEM((CH, B), jnp.int32),
        pltpu.VMEM((NBUF, B, D), dt),
        pltpu.VMEM_SHARED((NPAD, D), dt),
    ] + [pltpu.SemaphoreType.DMA] * NBUF + [
        pltpu.SemaphoreType.DMA,
    ]
    if with_deg:
        outs.append(jax.ShapeDtypeStruct((NC * NPAD, 16), jnp.float32))
        scratch += [
            pltpu.VMEM((B, 16), jnp.float32),
            pltpu.VMEM_SHARED((NPAD, 16), jnp.float32),
        ]

    def body(src_hbm, dst_hbm, x_hbm, z_hbm, *rest):
        if with_deg:
            (z16_hbm, ones_hbm, out_hbm, deg_hbm, srcA, dstA, srcB, dstB,
             rows_v, acc_sh, *sems, uno_v, deg_sh) = rest
        else:
            (out_hbm, srcA, dstA, srcB, dstB, rows_v, acc_sh, *sems) = rest
        gsems = sems[:NBUF]
        isem = sems[NBUF]
        c = lax.axis_index("c")
        s = lax.axis_index("s")
        wid = s * NC + c
        row0 = s * ROWS_PER_TILE


        def gather(sv, j, buf):
            pltpu.async_copy(x_hbm.at[sv.at[j]], rows_v.at[buf], gsems[buf])

        def gwait(sv, j, buf):
            pltpu.make_async_copy(x_hbm.at[sv.at[j]], rows_v.at[buf],
                                  gsems[buf]).wait()

        def scatter(dv, j, buf):
            pltpu.sync_copy(rows_v.at[buf], acc_sh.at[dv.at[j]], add=True)

        def iwait(dst_ref):
            pltpu.make_async_copy(src_hbm.at[wid, pl.ds(0, CH)],
                                  dst_ref, isem).wait()

        NITER = STEPS // (2 * CH)

        def half(i, sv, dv, first):
            for p in range(CH):
                b = p % NBUF
                gwait(sv, p, b)
                scatter(dv, p, b)
                if with_deg:
                    pltpu.sync_copy(uno_v, deg_sh.at[dv.at[p]], add=True)
                nxt = p + NBUF
                if nxt < CH:
                    gather(sv, nxt, b)
                elif first:
                    if nxt == CH:
                        iwait(srcB)
                        iwait(dstB)
                    gather(srcB, nxt - CH, b)
                else:
                    @pl.when(i + 1 < NITER)
                    def _():
                        if nxt == CH:
                            iwait(srcA)
                            iwait(dstA)
                        gather(srcA, nxt - CH, b)

        def iteration(i, carry):
            cA = 2 * i
            cB = 2 * i + 1
            half(i, srcA, dstA, True)
            @pl.when(i + 1 < NITER)
            def _():
                pltpu.async_copy(
                    src_hbm.at[wid, pl.ds((cA + 2) * CH, CH)], srcA, isem)
                pltpu.async_copy(
                    dst_hbm.at[wid, pl.ds((cA + 2) * CH, CH)], dstA, isem)
            half(i, srcB, dstB, False)
            @pl.when(i + 1 < NITER)
            def _():
                pltpu.async_copy(
                    src_hbm.at[wid, pl.ds((cB + 2) * CH, CH)], srcB, isem)
                pltpu.async_copy(
                    dst_hbm.at[wid, pl.ds((cB + 2) * CH, CH)], dstB, isem)
            return carry

        pltpu.sync_copy(src_hbm.at[wid, pl.ds(0, CH)], srcA)
        pltpu.async_copy(src_hbm.at[wid, pl.ds(CH, CH)], srcB, isem)
        pltpu.async_copy(dst_hbm.at[wid, pl.ds(CH, CH)], dstB, isem)
        for b in range(1, NBUF):
            gather(srcA, b, b)
        pltpu.sync_copy(dst_hbm.at[wid, pl.ds(0, CH)], dstA)

        pltpu.sync_copy(z_hbm, rows_v.at[0])
        if with_deg:
            pltpu.sync_copy(z16_hbm, uno_v)
        for i in range(ROWS_PER_TILE // B):
            pltpu.sync_copy(rows_v.at[0], acc_sh.at[pl.ds(row0 + i * B, B)])
            if with_deg:
                pltpu.sync_copy(uno_v, deg_sh.at[pl.ds(row0 + i * B, B)])
        if with_deg:
            pltpu.sync_copy(ones_hbm, uno_v)
        plsc.subcore_barrier()
        gather(srcA, 0, 0)
        lax.fori_loop(0, NITER, iteration, 0)

        plsc.subcore_barrier()
        out_base = c * NPAD + row0
        NDR = ROWS_PER_TILE // B
        for i in range(NDR):
            b = i % 2
            if i >= 2:
                pltpu.make_async_copy(
                    rows_v.at[b],
                    out_hbm.at[pl.ds(out_base + (i - 2) * B, B)],
                    gsems[b]).wait()
            pltpu.sync_copy(acc_sh.at[pl.ds(row0 + i * B, B)], rows_v.at[b])
            pltpu.async_copy(rows_v.at[b],
                             out_hbm.at[pl.ds(out_base + i * B, B)],
                             gsems[b])
            if with_deg:
                pltpu.sync_copy(deg_sh.at[pl.ds(row0 + i * B, B)], uno_v)
                pltpu.sync_copy(uno_v,
                                deg_hbm.at[pl.ds(out_base + i * B, B)])
        for i in range(NDR - 2, NDR):
            b = i % 2
            pltpu.make_async_copy(
                rows_v.at[b],
                out_hbm.at[pl.ds(out_base + i * B, B)],
                gsems[b]).wait()

    return pl.kernel(
        body, out_type=tuple(outs) if with_deg else outs[0],
        mesh=mesh, scratch_types=scratch,
        compiler_params=pltpu.CompilerParams(
            use_tc_tiling_on_sc=tc_tiling,
            disable_bounds_checks=True,
            disable_semaphore_checks=True))


_agg_xd = _make_agg(F, tc_tiling=False, dt=jnp.bfloat16, with_deg=True)
_agg_x = _make_agg(F, tc_tiling=False, dt=jnp.bfloat16)
_agg_g = _make_agg(CP, tc_tiling=False, dt=jnp.bfloat16)


BN = 1024


def _tc1_body(p_ref, d_ref, w_ref, b_ref, o_ref):
    inv = 1.0 / jnp.maximum(d_ref[0, :, 0] + d_ref[1, :, 0], 1.0)
    psum = p_ref[0].astype(jnp.float32) + p_ref[1].astype(jnp.float32)
    h = psum * inv[:, None]
    y = jnp.dot(h, w_ref[...], preferred_element_type=jnp.float32,
                precision=lax.Precision.HIGHEST)
    o_ref[...] = jnp.maximum(y + b_ref[...], 0.0).astype(jnp.bfloat16)


def _tc2_body(p_ref, d_ref, w1_ref, b1_ref, w2_ref, o_ref):
    inv = 1.0 / jnp.maximum(d_ref[0, :, 0] + d_ref[1, :, 0], 1.0)
    psum = p_ref[0].astype(jnp.float32) + p_ref[1].astype(jnp.float32)
    h = psum * inv[:, None]
    y = jnp.dot(h, w1_ref[...], preferred_element_type=jnp.float32,
                precision=lax.Precision.HIGHEST)
    y = jnp.maximum(y + b1_ref[...], 0.0)
    g = jnp.dot(y, w2_ref[...], preferred_element_type=jnp.float32,
                precision=lax.Precision.HIGHEST)
    o_ref[...] = g.astype(jnp.bfloat16)


def _tc3_body(p_ref, d_ref, b_ref, o_ref):
    inv = 1.0 / jnp.maximum(d_ref[0, :, 0] + d_ref[1, :, 0], 1.0)
    psum = p_ref[0].astype(jnp.float32) + p_ref[1].astype(jnp.float32)
    z = psum * inv[:, None] + b_ref[...]
    mask = lax.broadcasted_iota(jnp.int32, (BN, CP), 1) < C
    zm = jnp.where(mask, z, -jnp.inf)
    m = jnp.max(zm, axis=1, keepdims=True)
    e = jnp.where(mask, jnp.exp(z - m), 0.0)
    lse = jnp.log(jnp.sum(e, axis=1, keepdims=True))
    o_ref[...] = z - m - lse


def _pspec(d):
    return pl.BlockSpec((2, BN, d), lambda i: (0, i, 0))


def _dspec():
    return pl.BlockSpec((2, BN, 16), lambda i: (0, i, 0))


def _full(shape):
    nd = len(shape)
    return pl.BlockSpec(shape, lambda i: (0,) * nd)


def _tc1(p, d, w, b):
    return pl.pallas_call(
        _tc1_body,
        grid=(NPAD // BN,),
        in_specs=[_pspec(H), _dspec(), _full((F, H)), _full((1, H))],
        out_specs=pl.BlockSpec((BN, H), lambda i: (i, 0)),
        out_shape=jax.ShapeDtypeStruct((NPAD, H), jnp.bfloat16),
    )(p, d, w, b)


def _tc2(p, d, w1, b1, w2):
    return pl.pallas_call(
        _tc2_body,
        grid=(NPAD // BN,),
        in_specs=[_pspec(H), _dspec(), _full((H, H)), _full((1, H)),
                  _full((H, CP))],
        out_specs=pl.BlockSpec((BN, CP), lambda i: (i, 0)),
        out_shape=jax.ShapeDtypeStruct((NPAD, CP), jnp.bfloat16),
    )(p, d, w1, b1, w2)


def _tc3(p, d, b):
    return pl.pallas_call(
        _tc3_body,
        grid=(NPAD // BN,),
        in_specs=[_pspec(CP), _dspec(), _full((1, CP))],
        out_specs=pl.BlockSpec((BN, CP), lambda i: (i, 0)),
        out_shape=jax.ShapeDtypeStruct((NPAD, CP), jnp.float32),
    )(p, d, b)


def kernel(features, edge_index, W0, b0, W1, b1, W2, b2):
    ei = edge_index.astype(jnp.int32)
    pad = NW * EPW - E
    src = jnp.concatenate([ei[0], jnp.full((pad,), N, jnp.int32)])
    dst = jnp.concatenate([ei[1], jnp.full((pad,), N, jnp.int32)])
    src = src.reshape(NW, STEPS, B)
    dst = dst.reshape(NW, STEPS, B)

    x = jnp.zeros((NPAD, F), jnp.bfloat16).at[:N].set(
        features.astype(jnp.bfloat16))
    zF = jnp.zeros((B, F), jnp.bfloat16)
    zCP = jnp.zeros((B, CP), jnp.bfloat16)
    z16 = jnp.zeros((B, 16), jnp.float32)
    ones16 = jnp.ones((B, 16), jnp.float32)

    w2p = jnp.zeros((H, CP), jnp.float32).at[:, :C].set(W2)
    b2p = jnp.zeros((1, CP), jnp.float32).at[0, :C].set(b2)

    p0, deg = _agg_xd(src, dst, x, zF, z16, ones16)
    p0 = p0.reshape(NC, NPAD, F)
    deg = deg.reshape(NC, NPAD, 16)

    h1 = _tc1(p0, deg, W0, b0.reshape(1, H))
    q = _agg_x(src, dst, h1, zF).reshape(NC, NPAD, H)
    g = _tc2(q, deg, W1, b1.reshape(1, H), w2p)
    r = _agg_g(src, dst, g, zCP).reshape(NC, NPAD, CP)
    z = _tc3(r, deg, b2p)
    return z[:N, :C]
